# Initial kernel scaffold; baseline (speedup 1.0000x reference)
import jax
import jax.numpy as jnp
from jax.experimental import pallas as pl


def kernel(images, conv_w, conv_b, fc_w, fc_b, gamma, beta):
    raise NotImplementedError("write your pallas kernel here")


# speedup vs baseline: 3.9678x; 3.9678x over previous
import functools

import jax
import jax.numpy as jnp
from jax.experimental import pallas as pl
from jax.experimental.pallas import tpu as pltpu

EPS = 1e-5


def _conv_pool_kernel(x_ref, w_ref, cb_ref, pool_ref, p_ref, acc_ref,
                      *, h, w, c, g):
    feat = w_ref.shape[-1]

    for r in range(h + 2):
        for dx in range(3):
            piece = jnp.concatenate(
                [x_ref[gg, :, r, dx:dx + w] for gg in range(g)], axis=1)
            base = (r * 3 + dx) * c
            p_ref[base:base + c, :] = piece

    bias = cb_ref[...]
    for i in range(h):
        patch = p_ref[i * 3 * c:(i + 3) * 3 * c, :]
        d = jax.lax.dot_general(patch, w_ref[...],
                                (((0,), (0,)), ((), ())),
                                preferred_element_type=jnp.float32)
        a = jnp.maximum(d + bias, 0.0)
        if i == 0:
            acc_ref[...] = a
        else:
            acc_ref[...] = acc_ref[...] + a
    pool_ref[...] = jnp.sum(acc_ref[...].reshape(g, w, feat), axis=1)


def _fc_bn_kernel(pool_ref, fcw_ref, fcb_ref, gam_ref, bet_ref, o_ref,
                  *, inv_hw):
    feats = pool_ref[...] * inv_hw
    y = (jnp.dot(feats, fcw_ref[...], preferred_element_type=jnp.float32)
         + fcb_ref[...])
    mu = jnp.mean(y, axis=0, keepdims=True)
    yc = y - mu
    var = jnp.mean(yc * yc, axis=0, keepdims=True)
    o_ref[...] = gam_ref[...] * yc * jax.lax.rsqrt(var + EPS) + bet_ref[...]


def kernel(images, conv_w, conv_b, fc_w, fc_b, gamma, beta):
    n, c, h, w = images.shape
    feat = conv_w.shape[-1]
    embed = fc_w.shape[-1]

    g = 16
    while n % g:
        g //= 2

    xp = jnp.pad(images, ((0, 0), (0, 0), (1, 1), (1, 1))).astype(jnp.bfloat16)
    w9 = conv_w.reshape(9 * c, feat).astype(jnp.bfloat16)
    cb = conv_b.reshape(1, feat).astype(jnp.float32)

    pool = pl.pallas_call(
        functools.partial(_conv_pool_kernel, h=h, w=w, c=c, g=g),
        out_shape=jax.ShapeDtypeStruct((n, feat), jnp.float32),
        grid=(n // g,),
        in_specs=[
            pl.BlockSpec((g, c, h + 2, w + 2), lambda b: (b, 0, 0, 0)),
            pl.BlockSpec((9 * c, feat), lambda b: (0, 0)),
            pl.BlockSpec((1, feat), lambda b: (0, 0)),
        ],
        out_specs=pl.BlockSpec((g, feat), lambda b: (b, 0)),
        scratch_shapes=[
            pltpu.VMEM(((h + 2) * 3 * c, g * w), jnp.bfloat16),
            pltpu.VMEM((g * w, feat), jnp.float32),
        ],
        compiler_params=pltpu.CompilerParams(
            dimension_semantics=("parallel",)),
    )(xp, w9, cb)

    out = pl.pallas_call(
        functools.partial(_fc_bn_kernel, inv_hw=1.0 / (h * w)),
        out_shape=jax.ShapeDtypeStruct((n, embed), jnp.float32),
        grid=(1,),
        in_specs=[
            pl.BlockSpec((n, feat), lambda i: (0, 0)),
            pl.BlockSpec((feat, embed), lambda i: (0, 0)),
            pl.BlockSpec((1, embed), lambda i: (0, 0)),
            pl.BlockSpec((1, embed), lambda i: (0, 0)),
            pl.BlockSpec((1, embed), lambda i: (0, 0)),
        ],
        out_specs=pl.BlockSpec((n, embed), lambda i: (0, 0)),
    )(pool, fc_w, fc_b.reshape(1, embed), gamma.reshape(1, embed),
      beta.reshape(1, embed))
    return out


# speedup vs baseline: 6.3979x; 1.6125x over previous
import functools

import jax
import jax.numpy as jnp
from jax.experimental import pallas as pl
from jax.experimental.pallas import tpu as pltpu

EPS = 1e-5


def _conv_pool_kernel(x_ref, w_ref, cb_ref, pool_ref, p_ref, acc_ref,
                      *, h, w, c, g):
    feat = w_ref.shape[-1]
    m = g * w

    lane = jax.lax.broadcasted_iota(jnp.int32, (c, m), 1)
    left_edge = (lane % w) == 0
    right_edge = (lane % w) == (w - 1)
    zcol = jnp.zeros((c, 1), jnp.bfloat16)
    zero = jnp.zeros((), jnp.bfloat16)
    for r in range(h + 2):
        row = x_ref[r]
        base = r * 3 * c
        sr = jnp.concatenate([zcol, row[:, :m - 1]], axis=1)
        p_ref[base:base + c, :] = jnp.where(left_edge, zero, sr)
        p_ref[base + c:base + 2 * c, :] = row
        sl = jnp.concatenate([row[:, 1:], zcol], axis=1)
        p_ref[base + 2 * c:base + 3 * c, :] = jnp.where(right_edge, zero, sl)

    bias = cb_ref[...]
    for i in range(h):
        patch = p_ref[i * 3 * c:(i + 3) * 3 * c, :]
        d = jax.lax.dot_general(patch, w_ref[...],
                                (((0,), (0,)), ((), ())),
                                preferred_element_type=jnp.float32)
        a = jnp.maximum(d + bias, 0.0)
        if i == 0:
            acc_ref[...] = a
        else:
            acc_ref[...] = acc_ref[...] + a
    pool_ref[...] = jnp.sum(acc_ref[...].reshape(g, w, feat), axis=1)


def _fc_bn_kernel(pool_ref, fcw_ref, fcb_ref, gam_ref, bet_ref, o_ref,
                  *, inv_hw):
    feats = pool_ref[...] * inv_hw
    y = (jnp.dot(feats, fcw_ref[...], preferred_element_type=jnp.float32)
         + fcb_ref[...])
    mu = jnp.mean(y, axis=0, keepdims=True)
    yc = y - mu
    var = jnp.mean(yc * yc, axis=0, keepdims=True)
    o_ref[...] = gam_ref[...] * yc * jax.lax.rsqrt(var + EPS) + bet_ref[...]


def kernel(images, conv_w, conv_b, fc_w, fc_b, gamma, beta):
    n, c, h, w = images.shape
    feat = conv_w.shape[-1]
    embed = fc_w.shape[-1]

    g = 16
    while n % g:
        g //= 2

    xt = (jnp.pad(images, ((0, 0), (0, 0), (1, 1), (0, 0)))
          .transpose(2, 1, 0, 3).reshape(h + 2, c, n * w).astype(jnp.bfloat16))
    w9 = conv_w.reshape(9 * c, feat).astype(jnp.bfloat16)
    cb = conv_b.reshape(1, feat).astype(jnp.float32)

    pool = pl.pallas_call(
        functools.partial(_conv_pool_kernel, h=h, w=w, c=c, g=g),
        out_shape=jax.ShapeDtypeStruct((n, feat), jnp.float32),
        grid=(n // g,),
        in_specs=[
            pl.BlockSpec((h + 2, c, g * w), lambda b: (0, 0, b)),
            pl.BlockSpec((9 * c, feat), lambda b: (0, 0)),
            pl.BlockSpec((1, feat), lambda b: (0, 0)),
        ],
        out_specs=pl.BlockSpec((g, feat), lambda b: (b, 0)),
        scratch_shapes=[
            pltpu.VMEM(((h + 2) * 3 * c, g * w), jnp.bfloat16),
            pltpu.VMEM((g * w, feat), jnp.float32),
        ],
        compiler_params=pltpu.CompilerParams(
            dimension_semantics=("parallel",)),
    )(xt, w9, cb)

    out = pl.pallas_call(
        functools.partial(_fc_bn_kernel, inv_hw=1.0 / (h * w)),
        out_shape=jax.ShapeDtypeStruct((n, embed), jnp.float32),
        grid=(1,),
        in_specs=[
            pl.BlockSpec((n, feat), lambda i: (0, 0)),
            pl.BlockSpec((feat, embed), lambda i: (0, 0)),
            pl.BlockSpec((1, embed), lambda i: (0, 0)),
            pl.BlockSpec((1, embed), lambda i: (0, 0)),
            pl.BlockSpec((1, embed), lambda i: (0, 0)),
        ],
        out_specs=pl.BlockSpec((n, embed), lambda i: (0, 0)),
    )(pool, fc_w, fc_b.reshape(1, embed), gamma.reshape(1, embed),
      beta.reshape(1, embed))
    return out
